# baseline (speedup 1.0000x reference)
import functools

import jax
import jax.numpy as jnp
from jax.experimental import pallas as pl
from jax.experimental.pallas import tpu as pltpu

_H_PAD = 128
_BIAS_ROWS = 8
_N_ACTIONS = 64
_TB = 2048


def _round_up(x, m):
    return (x + m - 1) // m * m


def _mlp_kernel(obs_pad, x_ref, slab_ref, o_ref):
    base2 = obs_pad + _BIAS_ROWS
    base3 = base2 + _H_PAD + _BIAS_ROWS
    n_obs = x_ref.shape[-1]

    x = x_ref[...].astype(jnp.bfloat16)
    w1 = slab_ref[:n_obs, :].astype(jnp.bfloat16)
    h = jnp.dot(x, w1, preferred_element_type=jnp.float32)
    h = jnp.maximum(h + slab_ref[obs_pad:obs_pad + 1, :], 0.0)

    w2 = slab_ref[base2:base2 + _H_PAD, :].astype(jnp.bfloat16)
    h = jnp.dot(h.astype(jnp.bfloat16), w2, preferred_element_type=jnp.float32)
    h = jnp.maximum(h + slab_ref[base2 + _H_PAD:base2 + _H_PAD + 1, :], 0.0)

    w3 = slab_ref[base3:base3 + _H_PAD, :].astype(jnp.bfloat16)
    out = jnp.dot(h.astype(jnp.bfloat16), w3, preferred_element_type=jnp.float32)
    out = out + slab_ref[base3 + _H_PAD:base3 + _H_PAD + 1, :]
    o_ref[...] = out[:, :_N_ACTIONS]


@jax.jit
def kernel(x, slab):
    B, n_obs = x.shape
    obs_pad = _round_up(n_obs, 8)

    tb = min(_TB, _round_up(B, 8))
    b_pad = _round_up(B, tb)
    x_p = x if b_pad == B else jnp.pad(x, ((0, b_pad - B), (0, 0)))

    out = pl.pallas_call(
        functools.partial(_mlp_kernel, obs_pad),
        out_shape=jax.ShapeDtypeStruct((b_pad, _N_ACTIONS), jnp.float32),
        grid=(b_pad // tb,),
        in_specs=[
            pl.BlockSpec((tb, n_obs), lambda i: (i, 0)),
            pl.BlockSpec(slab.shape, lambda i: (0, 0)),
        ],
        out_specs=pl.BlockSpec((tb, _N_ACTIONS), lambda i: (i, 0)),
        compiler_params=pltpu.CompilerParams(
            dimension_semantics=("parallel",),
        ),
    )(x_p, slab)

    return out if b_pad == B else out[:B]


# speedup vs baseline: 1.3265x; 1.3265x over previous
import functools

import jax
import jax.numpy as jnp
from jax.experimental import pallas as pl
from jax.experimental.pallas import tpu as pltpu

_H_PAD = 128
_BIAS_ROWS = 8
_N_ACTIONS = 64
_TB = 8192


def _round_up(x, m):
    return (x + m - 1) // m * m


def _mlp_kernel(obs_pad, x_ref, slab_ref, o_ref):
    base2 = obs_pad + _BIAS_ROWS
    base3 = base2 + _H_PAD + _BIAS_ROWS
    n_obs = x_ref.shape[-1]

    x = x_ref[...].astype(jnp.bfloat16)
    w1 = slab_ref[:n_obs, :].astype(jnp.bfloat16)
    h = jnp.dot(x, w1, preferred_element_type=jnp.float32)
    h = jnp.maximum(h + slab_ref[obs_pad:obs_pad + 1, :], 0.0)

    w2 = slab_ref[base2:base2 + _H_PAD, :].astype(jnp.bfloat16)
    h = jnp.dot(h.astype(jnp.bfloat16), w2, preferred_element_type=jnp.float32)
    h = jnp.maximum(h + slab_ref[base2 + _H_PAD:base2 + _H_PAD + 1, :], 0.0)

    w3 = slab_ref[base3:base3 + _H_PAD, :].astype(jnp.bfloat16)
    out = jnp.dot(h.astype(jnp.bfloat16), w3, preferred_element_type=jnp.float32)
    out = out + slab_ref[base3 + _H_PAD:base3 + _H_PAD + 1, :]
    o_ref[...] = out[:, :_N_ACTIONS]


@jax.jit
def kernel(x, slab):
    B, n_obs = x.shape
    obs_pad = _round_up(n_obs, 8)

    tb = min(_TB, _round_up(B, 8))
    b_pad = _round_up(B, tb)
    x_p = x if b_pad == B else jnp.pad(x, ((0, b_pad - B), (0, 0)))

    out = pl.pallas_call(
        functools.partial(_mlp_kernel, obs_pad),
        out_shape=jax.ShapeDtypeStruct((b_pad, _N_ACTIONS), jnp.float32),
        grid=(b_pad // tb,),
        in_specs=[
            pl.BlockSpec((tb, n_obs), lambda i: (i, 0)),
            pl.BlockSpec(slab.shape, lambda i: (0, 0)),
        ],
        out_specs=pl.BlockSpec((tb, _N_ACTIONS), lambda i: (i, 0)),
        compiler_params=pltpu.CompilerParams(
            dimension_semantics=("parallel",),
        ),
    )(x_p, slab)

    return out if b_pad == B else out[:B]


# speedup vs baseline: 1.3663x; 1.0301x over previous
import functools

import jax
import jax.numpy as jnp
from jax.experimental import pallas as pl
from jax.experimental.pallas import tpu as pltpu

_H_PAD = 128
_BIAS_ROWS = 8
_N_ACTIONS = 64
_TB = 16384


def _round_up(x, m):
    return (x + m - 1) // m * m


def _mlp_kernel(obs_pad, x_ref, slab_ref, o_ref):
    base2 = obs_pad + _BIAS_ROWS
    base3 = base2 + _H_PAD + _BIAS_ROWS
    n_obs = x_ref.shape[-1]

    x = x_ref[...].astype(jnp.bfloat16)
    w1 = slab_ref[:n_obs, :].astype(jnp.bfloat16)
    h = jnp.dot(x, w1, preferred_element_type=jnp.float32)
    h = jnp.maximum(h + slab_ref[obs_pad:obs_pad + 1, :], 0.0)

    w2 = slab_ref[base2:base2 + _H_PAD, :].astype(jnp.bfloat16)
    h = jnp.dot(h.astype(jnp.bfloat16), w2, preferred_element_type=jnp.float32)
    h = jnp.maximum(h + slab_ref[base2 + _H_PAD:base2 + _H_PAD + 1, :], 0.0)

    w3 = slab_ref[base3:base3 + _H_PAD, :].astype(jnp.bfloat16)
    out = jnp.dot(h.astype(jnp.bfloat16), w3, preferred_element_type=jnp.float32)
    out = out + slab_ref[base3 + _H_PAD:base3 + _H_PAD + 1, :]
    o_ref[...] = out[:, :_N_ACTIONS]


@jax.jit
def kernel(x, slab):
    B, n_obs = x.shape
    obs_pad = _round_up(n_obs, 8)

    tb = min(_TB, _round_up(B, 8))
    b_pad = _round_up(B, tb)
    x_p = x if b_pad == B else jnp.pad(x, ((0, b_pad - B), (0, 0)))

    out = pl.pallas_call(
        functools.partial(_mlp_kernel, obs_pad),
        out_shape=jax.ShapeDtypeStruct((b_pad, _N_ACTIONS), jnp.float32),
        grid=(b_pad // tb,),
        in_specs=[
            pl.BlockSpec((tb, n_obs), lambda i: (i, 0)),
            pl.BlockSpec(slab.shape, lambda i: (0, 0)),
        ],
        out_specs=pl.BlockSpec((tb, _N_ACTIONS), lambda i: (i, 0)),
        compiler_params=pltpu.CompilerParams(
            dimension_semantics=("parallel",),
        ),
    )(x_p, slab)

    return out if b_pad == B else out[:B]
